# Initial kernel scaffold; baseline (speedup 1.0000x reference)
import jax
import jax.numpy as jnp
from jax.experimental import pallas as pl


def kernel(x, table):
    raise NotImplementedError("write your pallas kernel here")


# speedup vs baseline: 4.2472x; 4.2472x over previous
import functools

import jax
import jax.numpy as jnp
from jax import lax
from jax.experimental import pallas as pl
from jax.experimental.pallas import tpu as pltpu
from jax.experimental.pallas import tpu_sc as plsc

_CHUNK = 128
_K = 8


def _emb_call(num_cores, num_subcores, n_chunks, D):
    mesh = plsc.VectorSubcoreMesh(core_axis_name="c", subcore_axis_name="s")
    n_workers = num_cores * num_subcores
    B = n_workers * n_chunks * _CHUNK
    per_w = n_chunks * _CHUNK

    @functools.partial(
        pl.kernel,
        mesh=mesh,
        out_type=jax.ShapeDtypeStruct((B, D), jnp.float32),
        compiler_params=pltpu.CompilerParams(use_tc_tiling_on_sc=False),
        scratch_types=[
            pltpu.VMEM((n_chunks, _CHUNK), jnp.int32),
            pltpu.VMEM((_K, _CHUNK, D), jnp.float32),
            pltpu.SemaphoreType.DMA,
            pltpu.SemaphoreType.DMA,
        ],
    )
    def emb(idx_hbm, table_hbm, out_hbm, idx_v, rows_v, gsem, ssem):
        wid = lax.axis_index("s") * num_cores + lax.axis_index("c")
        pltpu.sync_copy(idx_hbm.at[wid], idx_v)
        base = wid * per_w

        def group(i, carry):
            g0 = i * _K
            gds = [
                pltpu.async_copy(
                    table_hbm.at[idx_v.at[g0 + b]], rows_v.at[b], gsem
                )
                for b in range(_K)
            ]
            sds = []
            for b in range(_K):
                gds[b].wait()
                sds.append(
                    pltpu.async_copy(
                        rows_v.at[b],
                        out_hbm.at[pl.ds(base + (g0 + b) * _CHUNK, _CHUNK)],
                        ssem,
                    )
                )
            for s in sds:
                s.wait()
            return carry

        lax.fori_loop(0, n_chunks // _K, group, 0)

    return emb


def kernel(x, table):
    R, S = x.shape
    V, D = table.shape
    B = R * S
    info = plsc.get_sparse_core_info()
    n_workers = info.num_cores * info.num_subcores
    n_chunks = B // (n_workers * _CHUNK)
    idx = x.reshape(n_workers, n_chunks, _CHUNK).astype(jnp.int32)
    out = _emb_call(info.num_cores, info.num_subcores, n_chunks, D)(idx, table)
    return out.reshape(R, S, D)


# speedup vs baseline: 4.2667x; 1.0046x over previous
import functools

import jax
import jax.numpy as jnp
from jax import lax
from jax.experimental import pallas as pl
from jax.experimental.pallas import tpu as pltpu
from jax.experimental.pallas import tpu_sc as plsc

_CHUNK = 128
_NB = 8
_G = 4


def _emb_call(num_cores, num_subcores, n_chunks, D):
    mesh = plsc.VectorSubcoreMesh(core_axis_name="c", subcore_axis_name="s")
    n_workers = num_cores * num_subcores
    B = n_workers * n_chunks * _CHUNK
    per_w = n_chunks * _CHUNK
    assert n_chunks % _NB == 0 and n_chunks >= 2 * _NB

    @functools.partial(
        pl.kernel,
        mesh=mesh,
        out_type=jax.ShapeDtypeStruct((B, D), jnp.float32),
        compiler_params=pltpu.CompilerParams(use_tc_tiling_on_sc=False),
        scratch_types=[
            pltpu.VMEM((n_chunks, _CHUNK), jnp.int32),
            pltpu.VMEM((_NB, _CHUNK, D), jnp.float32),
            pltpu.SemaphoreType.DMA((_NB,)),
            pltpu.SemaphoreType.DMA((_NB,)),
        ],
    )
    def emb(idx_hbm, table_hbm, out_hbm, idx_v, rows_v, gsems, ssems):
        wid = lax.axis_index("s") * num_cores + lax.axis_index("c")
        pltpu.sync_copy(idx_hbm.at[wid], idx_v)
        base = wid * per_w

        def gather(j, b):
            pltpu.async_copy(table_hbm.at[idx_v.at[j]], rows_v.at[b], gsems.at[b])

        def scatter(j, b):
            pltpu.async_copy(
                rows_v.at[b],
                out_hbm.at[pl.ds(base + j * _CHUNK, _CHUNK)],
                ssems.at[b],
            )

        def drain_g(b):
            pltpu.make_async_copy(
                table_hbm.at[pl.ds(0, _CHUNK)], rows_v.at[b], gsems.at[b]
            ).wait()

        def drain_s(b):
            pltpu.make_async_copy(
                rows_v.at[b], out_hbm.at[pl.ds(base, _CHUNK)], ssems.at[b]
            ).wait()

        def step(j, b, drain_scatter, prefetch):
            drain_g(b)
            scatter(j, b)
            if prefetch:
                bp = (b + _G) % _NB
                if drain_scatter:
                    drain_s(bp)
                gather(j + _G, bp)

        for j in range(_G):
            gather(j, j)
        for j in range(_NB):
            step(j, j, drain_scatter=(j + _G >= _NB), prefetch=True)

        def body(i, carry):
            j0 = i * _NB
            for b in range(_NB):
                step(j0 + b, b, drain_scatter=True, prefetch=True)
            return carry

        lax.fori_loop(1, n_chunks // _NB - 1, body, 0)

        j0 = n_chunks - _NB
        for b in range(_NB):
            step(j0 + b, b, drain_scatter=True, prefetch=(j0 + b + _G < n_chunks))
        for b in range(_NB):
            drain_s(b)

    return emb


def kernel(x, table):
    R, S = x.shape
    V, D = table.shape
    B = R * S
    info = plsc.get_sparse_core_info()
    n_workers = info.num_cores * info.num_subcores
    n_chunks = B // (n_workers * _CHUNK)
    idx = x.reshape(n_workers, n_chunks, _CHUNK).astype(jnp.int32)
    out = _emb_call(info.num_cores, info.num_subcores, n_chunks, D)(idx, table)
    return out.reshape(R, S, D)
